# baseline (speedup 1.0000x reference)
import jax
import jax.numpy as jnp
from jax.experimental import pallas as pl


def _pos_kernel(row_ref, col_ref, out_ref):
    h, w = out_ref.shape[2], out_ref.shape[3]
    f = row_ref.shape[1]
    col_t = col_ref[...].T
    row_t = row_ref[...].T
    top = jnp.broadcast_to(col_t[:, None, :], (f, h, w))
    bot = jnp.broadcast_to(row_t[:, :, None], (f, h, w))
    out_ref[0] = jnp.concatenate([top, bot], axis=0)


def kernel(x, row_embed, col_embed):
    b = x.shape[0]
    h, w = x.shape[-2], x.shape[-1]
    f = row_embed.shape[1]
    return pl.pallas_call(
        _pos_kernel,
        grid=(b,),
        in_specs=[
            pl.BlockSpec((h, f), lambda i: (0, 0)),
            pl.BlockSpec((w, f), lambda i: (0, 0)),
        ],
        out_specs=pl.BlockSpec((1, 2 * f, h, w), lambda i: (i, 0, 0, 0)),
        out_shape=jax.ShapeDtypeStruct((b, 2 * f, h, w), row_embed.dtype),
    )(row_embed, col_embed)


# speedup vs baseline: 1.6708x; 1.6708x over previous
import jax
import jax.numpy as jnp
from jax.experimental import pallas as pl


def _pos_kernel(row_ref, col_ref, out_ref):
    hw = out_ref.shape[2]
    f = row_ref.shape[1]
    h = row_ref.shape[0]
    w = col_ref.shape[0]
    col_t = col_ref[...].T
    row_t = row_ref[...].T
    top = jnp.broadcast_to(col_t[:, None, :], (f, h, w)).reshape(f, hw)
    bot = jnp.broadcast_to(row_t[:, :, None], (f, h, w)).reshape(f, hw)
    out_ref[0] = jnp.concatenate([top, bot], axis=0)


def kernel(x, row_embed, col_embed):
    b = x.shape[0]
    h, w = x.shape[-2], x.shape[-1]
    f = row_embed.shape[1]
    flat = pl.pallas_call(
        _pos_kernel,
        grid=(b,),
        in_specs=[
            pl.BlockSpec((h, f), lambda i: (0, 0)),
            pl.BlockSpec((w, f), lambda i: (0, 0)),
        ],
        out_specs=pl.BlockSpec((1, 2 * f, h * w), lambda i: (i, 0, 0)),
        out_shape=jax.ShapeDtypeStruct((b, 2 * f, h * w), row_embed.dtype),
    )(row_embed, col_embed)
    return flat.reshape(b, 2 * f, h, w)
